# baseline (speedup 1.0000x reference)
import functools

import jax
import jax.numpy as jnp
import numpy as np
from jax import lax
from jax.experimental import pallas as pl
from jax.experimental.pallas import tpu as pltpu
from jax.experimental.pallas import tpu_sc as plsc

_D = 64
_SEQ = 200
_CHUNK = 100
_B = 1024
_NC, _NS = 2, 16
_NW = _NC * _NS
_BPW = _B // _NW
_XROWS = (_B * _SEQ // _CHUNK) // _NW


def _pos_embed(J, D):
    pos = jnp.arange(J, dtype=jnp.float32)[:, None]
    i = jnp.arange(0, D, 2, dtype=jnp.float32)
    i = jnp.exp(-(i / D) * np.log(10000.0))
    ang = pos * i[None, :]
    pe = jnp.zeros((J, D), dtype=jnp.float32)
    pe = pe.at[:, 0::2].set(jnp.sin(ang))
    pe = pe.at[:, 1::2].set(jnp.cos(ang))
    return pe


def _sc_embed(x2d, pe3, W):
    mesh = plsc.VectorSubcoreMesh(core_axis_name="c", subcore_axis_name="s")

    @functools.partial(
        pl.kernel,
        out_type=jax.ShapeDtypeStruct((_B, 2, _CHUNK, _D), jnp.float32),
        mesh=mesh,
        compiler_params=pltpu.CompilerParams(use_tc_tiling_on_sc=False),
        scratch_types=[
            pltpu.VMEM((_XROWS, _CHUNK), jnp.int32),
            pltpu.VMEM((2, _CHUNK, _D), jnp.float32),
            pltpu.VMEM((2, 2, _CHUNK, _D), jnp.float32),
            pltpu.SemaphoreType.DMA,
            pltpu.SemaphoreType.DMA,
            pltpu.SemaphoreType.DMA,
            pltpu.SemaphoreType.DMA,
        ],
    )
    def k(x_hbm, pe_hbm, w_hbm, out_hbm, idx_v, pe_v, buf_v, g0, g1, w0, w1):
        wid = lax.axis_index("s") * _NC + lax.axis_index("c")
        pltpu.sync_copy(x_hbm.at[pl.ds(wid * _XROWS, _XROWS)], idx_v)
        pltpu.sync_copy(pe_hbm, pe_v)

        gsem = (g0, g1)
        wsem = (w0, w1)

        def start_gather(t, s):
            c = 2 * t
            pltpu.async_copy(w_hbm.at[idx_v.at[c]], buf_v.at[s, 0], gsem[s])
            pltpu.async_copy(w_hbm.at[idx_v.at[c + 1]], buf_v.at[s, 1], gsem[s])

        def wait_gather(s):
            pltpu.make_async_copy(out_hbm.at[0], buf_v.at[s], gsem[s]).wait()

        def start_write(b, s):
            pltpu.async_copy(buf_v.at[s], out_hbm.at[b], wsem[s])

        def wait_write(s):
            pltpu.make_async_copy(buf_v.at[s], out_hbm.at[0], wsem[s]).wait()

        def compute(s):
            def row(r, carry):
                for h in range(2):
                    for kk in range(_D // 16):
                        sl = pl.ds(kk * 16, 16)
                        v = buf_v[s, h, r, sl] * 8.0 + pe_v[h, r, sl]
                        buf_v[s, h, r, sl] = v
                return carry
            lax.fori_loop(0, _CHUNK, row, 0)

        start_gather(0, 0)

        def pair(p, carry):
            for s in range(2):
                t = 2 * p + s
                b = wid * _BPW + t
                if s == 0:
                    @pl.when(p >= 1)
                    def _w():
                        wait_write(1)
                    start_gather(t + 1, 1)
                else:
                    wait_write(0)
                    @pl.when(p < _BPW // 2 - 1)
                    def _g():
                        start_gather(t + 1, 0)
                wait_gather(s)
                compute(s)
                start_write(b, s)
            return carry

        lax.fori_loop(0, _BPW // 2, pair, 0)
        wait_write(1)

    return k(x2d, pe3, W)


def kernel(x, W):
    x2d = x.reshape(_B * _SEQ // _CHUNK, _CHUNK).astype(jnp.int32)
    pe3 = _pos_embed(_SEQ, _D).reshape(2, _CHUNK, _D)
    out = _sc_embed(x2d, pe3, W)
    return out.reshape(_B, _SEQ, _D)
